# Initial kernel scaffold; baseline (speedup 1.0000x reference)
import jax
import jax.numpy as jnp
from jax.experimental import pallas as pl


def kernel(indexes, tables):
    raise NotImplementedError("write your pallas kernel here")


# speedup vs baseline: 1.0901x; 1.0901x over previous
import jax
import jax.numpy as jnp
from jax import lax
from jax.experimental import pallas as pl
from jax.experimental.pallas import tpu as pltpu
from jax.experimental.pallas import tpu_sc as plsc

_NUM_FIELDS = 26
_VOCAB = 100000
_DIM = 64
_BATCH = 16384

_NC = 2
_NS = 16
_NW = _NC * _NS
_L = 16

_B_TOTAL = _BATCH * _NUM_FIELDS
_B_PER_W = _B_TOTAL // _NW
_CHUNK = 512
_N_CHUNKS = _B_PER_W // _CHUNK


def _gather_body(idx_hbm, tab_hbm, out_hbm, idx_v, rows_v, sem_g):
    wid = lax.axis_index("s") * _NC + lax.axis_index("c")
    base = wid * _B_PER_W

    pltpu.sync_copy(idx_hbm.at[pl.ds(base, _B_PER_W)], idx_v)

    def flatten_body(v, _):
        j0 = v * _L
        pos = j0 + lax.iota(jnp.int32, _L)
        field = lax.rem(pos, _NUM_FIELDS)
        idx_v[pl.ds(j0, _L)] = idx_v[pl.ds(j0, _L)] + field * _VOCAB
        return ()

    lax.fori_loop(0, _B_PER_W // _L, flatten_body, ())

    def chunk_body(c, _):
        cbase = c * _CHUNK
        pltpu.async_copy(
            tab_hbm.at[idx_v.at[pl.ds(cbase, _CHUNK)]], rows_v, sem_g
        ).wait()
        pltpu.sync_copy(rows_v, out_hbm.at[pl.ds(base + cbase, _CHUNK)])
        return ()

    lax.fori_loop(0, _N_CHUNKS, chunk_body, ())


_sc_gather = pl.kernel(
    _gather_body,
    out_type=jax.ShapeDtypeStruct((_B_TOTAL, _DIM), jnp.float32),
    mesh=plsc.VectorSubcoreMesh(core_axis_name="c", subcore_axis_name="s"),
    scratch_types=[
        pltpu.VMEM((_B_PER_W,), jnp.int32),
        pltpu.VMEM((_CHUNK, _DIM), jnp.float32),
        pltpu.SemaphoreType.DMA,
    ],
    compiler_params=pltpu.CompilerParams(use_tc_tiling_on_sc=False),
)


def kernel(indexes, tables):
    idx_flat = indexes.reshape(_B_TOTAL).astype(jnp.int32)
    tab_flat = tables.reshape(_NUM_FIELDS * _VOCAB, _DIM)
    out = _sc_gather(idx_flat, tab_flat)
    return out.reshape(_BATCH, _NUM_FIELDS * _DIM)


# speedup vs baseline: 1.0999x; 1.0090x over previous
import jax
import jax.numpy as jnp
from jax import lax
from jax.experimental import pallas as pl
from jax.experimental.pallas import tpu as pltpu
from jax.experimental.pallas import tpu_sc as plsc

_NUM_FIELDS = 26
_VOCAB = 100000
_DIM = 64
_BATCH = 16384

_NC = 2
_NS = 16
_NW = _NC * _NS
_L = 16

_B_TOTAL = _BATCH * _NUM_FIELDS
_B_PER_W = _B_TOTAL // _NW
_CHUNK = 256
_N_CHUNKS = _B_PER_W // _CHUNK
_NBUF = 4


def _gather_body(idx_hbm, tab_hbm, out_hbm, idx_v, rows_v, *sems):
    gsem = sems[:_NBUF]
    wsem = sems[_NBUF:]
    wid = lax.axis_index("s") * _NC + lax.axis_index("c")
    base = wid * _B_PER_W

    pltpu.sync_copy(idx_hbm.at[pl.ds(base, _B_PER_W)], idx_v)

    def flatten_body(v, _):
        j0 = v * _L
        pos = j0 + lax.iota(jnp.int32, _L)
        field = lax.rem(pos, _NUM_FIELDS)
        idx_v[pl.ds(j0, _L)] = idx_v[pl.ds(j0, _L)] + field * _VOCAB
        return ()

    lax.fori_loop(0, _B_PER_W // _L, flatten_body, ())

    def fire_gather(c, b):
        pltpu.async_copy(
            tab_hbm.at[idx_v.at[pl.ds(c * _CHUNK, _CHUNK)]],
            rows_v.at[b],
            gsem[b],
        )

    def fire_write(c, b):
        pltpu.async_copy(
            rows_v.at[b],
            out_hbm.at[pl.ds(base + c * _CHUNK, _CHUNK)],
            wsem[b],
        )

    def wait_gather(c, b):
        pltpu.make_async_copy(
            tab_hbm.at[idx_v.at[pl.ds(c * _CHUNK, _CHUNK)]],
            rows_v.at[b],
            gsem[b],
        ).wait()

    def wait_write(c, b):
        pltpu.make_async_copy(
            rows_v.at[b],
            out_hbm.at[pl.ds(base + c * _CHUNK, _CHUNK)],
            wsem[b],
        ).wait()

    for b in range(_NBUF):
        fire_gather(b, b)

    @pl.loop(0, _N_CHUNKS - _NBUF, step=_NBUF)
    def _steady(c0):
        for b in range(_NBUF):
            c = c0 + b
            wait_gather(c, b)
            fire_write(c, b)
            wait_write(c, b)
            fire_gather(c + _NBUF, b)

    for b in range(_NBUF):
        c = _N_CHUNKS - _NBUF + b
        wait_gather(c, b)
        fire_write(c, b)
    for b in range(_NBUF):
        c = _N_CHUNKS - _NBUF + b
        wait_write(c, b)


_sc_gather = pl.kernel(
    _gather_body,
    out_type=jax.ShapeDtypeStruct((_B_TOTAL, _DIM), jnp.float32),
    mesh=plsc.VectorSubcoreMesh(core_axis_name="c", subcore_axis_name="s"),
    scratch_types=(
        [
            pltpu.VMEM((_B_PER_W,), jnp.int32),
            pltpu.VMEM((_NBUF, _CHUNK, _DIM), jnp.float32),
        ]
        + [pltpu.SemaphoreType.DMA] * (2 * _NBUF)
    ),
    compiler_params=pltpu.CompilerParams(use_tc_tiling_on_sc=False),
)


def kernel(indexes, tables):
    idx_flat = indexes.reshape(_B_TOTAL).astype(jnp.int32)
    tab_flat = tables.reshape(_NUM_FIELDS * _VOCAB, _DIM)
    out = _sc_gather(idx_flat, tab_flat)
    return out.reshape(_BATCH, _NUM_FIELDS * _DIM)
